# baseline (speedup 1.0000x reference)
import functools
import math

import jax
import jax.numpy as jnp
from jax import lax
from jax.experimental import pallas as pl
from jax.experimental.pallas import tpu as pltpu
from jax.experimental.pallas import tpu_sc as plsc

N = 1000000
H, W = 1024, 2048
HW = H * W

NW = 32
C = 2048
N_PAD = 1 << 20
PW = N_PAD // NW
KCH = PW // C

BR = 32
NR = N_PAD // 1024


def _proj_body(rt_ref, xyz_ref, iq_ref, wq_ref):
    i = pl.program_id(0)
    x = xyz_ref[0]
    y = xyz_ref[1]
    z = xyz_ref[2]
    t0 = rt_ref[0, 9]
    t1 = rt_ref[0, 10]
    t2 = rt_ref[0, 11]
    px = x - t0
    py = y - t1
    pz = z - t2
    nx = rt_ref[0, 0] * px + rt_ref[0, 1] * py + rt_ref[0, 2] * pz
    ny = rt_ref[0, 3] * px + rt_ref[0, 4] * py + rt_ref[0, 5] * pz
    nz = rt_ref[0, 6] * px + rt_ref[0, 7] * py + rt_ref[0, 8] * pz

    theta = jnp.arctan2(jnp.sqrt(nx * nx + ny * ny), nz)
    phi = jnp.arctan2(ny, nx) + jnp.float32(math.pi)
    c0 = 2.0 * (1.0 - phi * jnp.float32(1.0 / (2.0 * math.pi))) - 1.0
    c1 = 2.0 * (theta * jnp.float32(1.0 / math.pi)) - 1.0
    xp = (c0 + 1.0) * jnp.float32(W / 2.0) - 0.5
    yp = (c1 + 1.0) * jnp.float32(H / 2.0) - 0.5

    x0 = jnp.floor(xp)
    y0 = jnp.floor(yp)
    wx1 = xp - x0
    wx0 = 1.0 - wx1
    wy1 = yp - y0
    wy0 = 1.0 - wy1

    row = i * BR + lax.broadcasted_iota(jnp.int32, (BR, 1024), 0)
    pos = row * 1024 + lax.broadcasted_iota(jnp.int32, (BR, 1024), 1)
    inb = pos < N

    corners = (
        (x0, y0, wx0 * wy0),
        (x0 + 1.0, y0, wx1 * wy0),
        (x0, y0 + 1.0, wx0 * wy1),
        (x0 + 1.0, y0 + 1.0, wx1 * wy1),
    )
    for k, (xi, yi, wk) in enumerate(corners):
        valid = ((xi >= 0.0) & (xi <= jnp.float32(W - 1))
                 & (yi >= 0.0) & (yi <= jnp.float32(H - 1)) & inb)
        xc = jnp.clip(xi, 0.0, jnp.float32(W - 1)).astype(jnp.int32)
        yc = jnp.clip(yi, 0.0, jnp.float32(H - 1)).astype(jnp.int32)
        idx = yc * W + xc
        iq_ref[k] = jnp.where(valid, idx, jnp.int32(HW))
        wq_ref[k] = wk


def _project(rt, x3):
    grid = (NR // BR,)
    return pl.pallas_call(
        _proj_body,
        grid=grid,
        in_specs=[
            pl.BlockSpec((1, 16), lambda i: (0, 0)),
            pl.BlockSpec((3, BR, 1024), lambda i: (0, i, 0)),
        ],
        out_specs=[
            pl.BlockSpec((4, BR, 1024), lambda i: (0, i, 0)),
            pl.BlockSpec((4, BR, 1024), lambda i: (0, i, 0)),
        ],
        out_shape=[
            jax.ShapeDtypeStruct((4, NR, 1024), jnp.int32),
            jax.ShapeDtypeStruct((4, NR, 1024), jnp.float32),
        ],
    )(rt, x3)


def _sc_body(table_hbm, iq_hbm, gath_hbm,
             idx0, idx1, idx2, idx3, rows0, rows1, rows2, rows3, sem):
    wid = lax.axis_index("s") * 2 + lax.axis_index("c")
    idxs = [idx0, idx1, idx2, idx3]
    rows = [rows0, rows1, rows2, rows3]

    def chunk_body(g, carry):
        base = wid * PW + g * C
        for k in range(4):
            pltpu.sync_copy(iq_hbm.at[k, pl.ds(base, C)], idxs[k])
        handles = [
            pltpu.async_copy(table_hbm.at[idxs[k]], rows[k], sem)
            for k in range(4)
        ]
        for h in handles:
            h.wait()
        for k in range(4):
            pltpu.sync_copy(rows[k], gath_hbm.at[k, pl.ds(base, C)])
        return carry

    lax.fori_loop(0, KCH, chunk_body, 0)


_sc_gather = functools.partial(
    pl.kernel,
    mesh=plsc.VectorSubcoreMesh(core_axis_name="c", subcore_axis_name="s"),
    out_type=jax.ShapeDtypeStruct((4, N_PAD, 4), jnp.float32),
    compiler_params=pltpu.CompilerParams(use_tc_tiling_on_sc=False),
    scratch_types=[
        pltpu.VMEM((C,), jnp.int32),
        pltpu.VMEM((C,), jnp.int32),
        pltpu.VMEM((C,), jnp.int32),
        pltpu.VMEM((C,), jnp.int32),
        pltpu.VMEM((C, 4), jnp.float32),
        pltpu.VMEM((C, 4), jnp.float32),
        pltpu.VMEM((C, 4), jnp.float32),
        pltpu.VMEM((C, 4), jnp.float32),
        pltpu.SemaphoreType.DMA,
    ],
)(_sc_body)


def _loss_body(gath_ref, wq_ref, rgb_ref, pcd_ref, num_ref, den_ref):
    i = pl.program_id(0)
    w = [wq_ref[k] for k in range(4)]
    sample = []
    for ch in range(4):
        acc = None
        for k in range(4):
            term = gath_ref[4 * k + ch] * w[k]
            acc = term if acc is None else acc + term
        sample.append(acc)
    sr, sg, sb, sw = sample
    dr = sr - rgb_ref[0]
    dg = sg - rgb_ref[1]
    db = sb - rgb_ref[2]
    s = dr * dr + dg * dg + db * db
    raw = 0.5 * (sw + pcd_ref[...]) * jnp.sqrt(s)
    mask = (sr != 0.0) | (sg != 0.0) | (sb != 0.0)
    numv = jnp.where(mask, raw, 0.0)
    denv = jnp.where(mask, 1.0, 0.0)
    nsum = jnp.sum(numv, axis=0, keepdims=True)
    dsum = jnp.sum(denv, axis=0, keepdims=True)

    @pl.when(i == 0)
    def _():
        num_ref[...] = jnp.zeros_like(num_ref)
        den_ref[...] = jnp.zeros_like(den_ref)

    num_ref[...] += nsum
    den_ref[...] += dsum


def _loss(gath3, wq3, rgb3, pcd3):
    grid = (NR // BR,)
    return pl.pallas_call(
        _loss_body,
        grid=grid,
        in_specs=[
            pl.BlockSpec((16, BR, 1024), lambda i: (0, i, 0)),
            pl.BlockSpec((4, BR, 1024), lambda i: (0, i, 0)),
            pl.BlockSpec((3, BR, 1024), lambda i: (0, i, 0)),
            pl.BlockSpec((BR, 1024), lambda i: (i, 0)),
        ],
        out_specs=[
            pl.BlockSpec((1, 1024), lambda i: (0, 0)),
            pl.BlockSpec((1, 1024), lambda i: (0, 0)),
        ],
        out_shape=[
            jax.ShapeDtypeStruct((1, 1024), jnp.float32),
            jax.ShapeDtypeStruct((1, 1024), jnp.float32),
        ],
    )(gath3, wq3, rgb3, pcd3)


def kernel(xyz, rgb, img, img_weight, pcd_weight, translation, yaw, pitch, roll):
    f32 = jnp.float32
    t0 = jnp.zeros(1, dtype=f32)
    t1 = jnp.ones(1, dtype=f32)
    RX = jnp.stack([jnp.stack([t1, t0, t0]),
                    jnp.stack([t0, jnp.cos(roll), -jnp.sin(roll)]),
                    jnp.stack([t0, jnp.sin(roll), jnp.cos(roll)])]).reshape(3, 3)
    RY = jnp.stack([jnp.stack([jnp.cos(pitch), t0, jnp.sin(pitch)]),
                    jnp.stack([t0, t1, t0]),
                    jnp.stack([-jnp.sin(pitch), t0, jnp.cos(pitch)])]).reshape(3, 3)
    RZ = jnp.stack([jnp.stack([jnp.cos(yaw), -jnp.sin(yaw), t0]),
                    jnp.stack([jnp.sin(yaw), jnp.cos(yaw), t0]),
                    jnp.stack([t0, t0, t1])]).reshape(3, 3)
    R = jnp.matmul(jnp.matmul(RZ, RY), RX)
    rt = jnp.concatenate(
        [R.reshape(9), translation.reshape(3), jnp.zeros((4,), f32)]
    ).reshape(1, 16)

    xyzp = jnp.pad(xyz.T, ((0, 0), (0, N_PAD - N)))
    x3 = xyzp.reshape(3, NR, 1024)
    rgb3 = jnp.pad(rgb.T, ((0, 0), (0, N_PAD - N))).reshape(3, NR, 1024)
    pcd3 = jnp.pad(pcd_weight, (0, N_PAD - N)).reshape(NR, 1024)
    table = jnp.concatenate(
        [img.reshape(HW, 3), img_weight.reshape(HW, 1)], axis=1)
    table = jnp.pad(table, ((0, 1), (0, 0)))

    iq3, wq3 = _project(rt, x3)
    iq = iq3.reshape(4, N_PAD)

    gath = _sc_gather(table, iq)
    gath3 = gath.transpose(0, 2, 1).reshape(16, NR, 1024)

    nump, denp = _loss(gath3, wq3, rgb3, pcd3)
    return nump.sum() / denp.sum()
